# Initial kernel scaffold; baseline (speedup 1.0000x reference)
import jax
import jax.numpy as jnp
from jax.experimental import pallas as pl


def kernel(x, time_table, person_table):
    raise NotImplementedError("write your pallas kernel here")


# speedup vs baseline: 8.7915x; 8.7915x over previous
import functools

import jax
import jax.numpy as jnp
from jax.experimental import pallas as pl
from jax.experimental.pallas import tpu as pltpu

D_MODEL = 128
HALF = 64
SEQ_TILE = 128


def _renorm(rows, max_norm=1.0):
    norm = jnp.sqrt(jnp.sum(rows * rows, axis=-1, keepdims=True))
    scale = jnp.where(norm > max_norm, max_norm / (norm + 1e-7), 1.0)
    return rows * scale


def _body(time_ref, person_ref, x_ref, out_ref):
    t = _renorm(time_ref[...])
    p = _renorm(person_ref[...])

    rows = jax.lax.broadcasted_iota(jnp.int32, (HALF, D_MODEL), 0)
    cols = jax.lax.broadcasted_iota(jnp.int32, (HALF, D_MODEL), 1)
    e_even = (cols == 2 * rows).astype(jnp.float32)
    e_odd = (cols == 2 * rows + 1).astype(jnp.float32)
    t_full = jnp.dot(t, e_even, preferred_element_type=jnp.float32)
    p_full = jnp.dot(p, e_odd, preferred_element_type=jnp.float32)

    out_ref[...] = (
        x_ref[...] + t_full[None, :, None, :] + p_full[None, None, :, :]
    )


@jax.jit
def kernel(x, time_table, person_table):
    B, S, N, D = x.shape
    n_s = S // SEQ_TILE
    grid = (n_s, B)
    return pl.pallas_call(
        _body,
        grid=grid,
        in_specs=[
            pl.BlockSpec((SEQ_TILE, HALF), lambda s, b: (s, 0)),
            pl.BlockSpec((N, HALF), lambda s, b: (0, 0)),
            pl.BlockSpec((1, SEQ_TILE, N, D), lambda s, b: (b, s, 0, 0)),
        ],
        out_specs=pl.BlockSpec((1, SEQ_TILE, N, D), lambda s, b: (b, s, 0, 0)),
        out_shape=jax.ShapeDtypeStruct(x.shape, x.dtype),
    )(time_table, person_table, x)
